# baseline (speedup 1.0000x reference)
import functools

import jax
import jax.numpy as jnp
from jax import lax
from jax.experimental import pallas as pl
from jax.experimental.pallas import tpu as pltpu
from jax.experimental.pallas import tpu_sc as plsc

_N_NODES = 10000
_D = 128
_N_POS = 320000
_N_EDGES = 2 * _N_POS
_NW = 32
_PER_W = _N_EDGES // _NW
_B = 80
_NCHUNK = _PER_W // _B
_L = 16
_KD = _D // _L
_DW = _D // 2
_KD32 = _DW // _L


def _sc_scores_body(emb_hbm, edges_hbm, out_hbm,
                    idx0s, idx0d, idx1s, idx1d,
                    rows0s, rows0d, rows1s, rows1d,
                    scores0, scores1,
                    sg0, sg1, si0, si1, so0, so1):
    wid = lax.axis_index("s") * 2 + lax.axis_index("c")
    w_base = wid * _PER_W

    idx = ((idx0s, idx0d), (idx1s, idx1d))
    rows = ((rows0s, rows0d), (rows1s, rows1d))
    scores = (scores0, scores1)
    sg = (sg0, sg1)
    si = (si0, si1)
    so = (so0, so1)

    def edge_base(c):
        return w_base + c * _B

    def idx_start(c, p):
        base = edge_base(c)
        pltpu.async_copy(edges_hbm.at[pl.ds(base, _B)], idx[p][0], si[p])
        pltpu.async_copy(
            edges_hbm.at[pl.ds(_N_EDGES + base, _B)], idx[p][1], si[p])

    def idx_wait(p):
        base = edge_base(0)
        pltpu.make_async_copy(
            edges_hbm.at[pl.ds(base, _B)], idx[p][0], si[p]).wait()
        pltpu.make_async_copy(
            edges_hbm.at[pl.ds(base, _B)], idx[p][1], si[p]).wait()

    def gather_start(p):
        pltpu.async_copy(emb_hbm.at[idx[p][0]], rows[p][0], sg[p])
        pltpu.async_copy(emb_hbm.at[idx[p][1]], rows[p][1], sg[p])

    def gather_wait(p):
        pltpu.make_async_copy(emb_hbm.at[idx[p][0]], rows[p][0], sg[p]).wait()
        pltpu.make_async_copy(emb_hbm.at[idx[p][1]], rows[p][1], sg[p]).wait()

    def out_start(c, p):
        pltpu.async_copy(
            scores[p].at[pl.ds(0, _B)], out_hbm.at[pl.ds(edge_base(c), _B)],
            so[p])

    def out_wait(p):
        pltpu.make_async_copy(
            scores[p].at[pl.ds(0, _B)], out_hbm.at[pl.ds(0, _B)],
            so[p]).wait()

    def compute(p):
        src_rows, dst_rows = rows[p]
        sc = scores[p]
        unroll = 16
        lane15 = lax.iota(jnp.int32, _L) == (_L - 1)

        def loads(i):
            return ([src_rows[i, pl.ds(k * _L, _L)] for k in range(_KD32)]
                    + [dst_rows[i, pl.ds(k * _L, _L)] for k in range(_KD32)])

        def bc(x):
            return plsc.bitcast(x, jnp.bfloat16)

        def chain_with_loads(i, cur, nxt_i):
            a0, a1, a2, a3, b0, b1, b2, b3 = cur
            n = [None] * 8
            n[0] = src_rows[nxt_i, pl.ds(0, _L)]
            m0 = bc(a0) * bc(b0)
            n[1] = src_rows[nxt_i, pl.ds(_L, _L)]
            m1 = bc(a1) * bc(b1)
            n[2] = src_rows[nxt_i, pl.ds(2 * _L, _L)]
            m2 = bc(a2) * bc(b2)
            n[3] = src_rows[nxt_i, pl.ds(3 * _L, _L)]
            m3 = bc(a3) * bc(b3)
            n[4] = dst_rows[nxt_i, pl.ds(0, _L)]
            t0 = m0 + m1
            n[5] = dst_rows[nxt_i, pl.ds(_L, _L)]
            t1 = m2 + m3
            n[6] = dst_rows[nxt_i, pl.ds(2 * _L, _L)]
            acc = t0 + t1
            n[7] = dst_rows[nxt_i, pl.ds(3 * _L, _L)]
            even, odd = plsc.unpack(acc, format=plsc.PackFormat.INTERLEAVED)
            tot = jnp.cumsum(even + odd)
            plsc.store_compressed(sc.at[pl.ds(i, _L)], tot, mask=lane15)
            return n

        def group(g, cur):
            for j in range(unroll):
                i = g * unroll + j
                cur = chain_with_loads(i, cur, jnp.minimum(i + 1, _B - 1))
            return tuple(cur)

        lax.fori_loop(0, _B // unroll, group, tuple(loads(0)))

    idx_start(0, 0)
    idx_wait(0)
    gather_start(0)
    idx_start(1, 1)

    def pair_body(it, carry):
        for p in (0, 1):
            q = 1 - p
            c = it * 2 + p
            gather_wait(p)
            idx_wait(q)
            gather_start(q)
            idx_start(jnp.minimum(c + 2, _NCHUNK - 1), p)

            @pl.when(it > 0)
            def _():
                out_wait(p)
            compute(p)
            out_start(c, p)
        return carry

    lax.fori_loop(0, _NCHUNK // 2, pair_body, 0)

    gather_wait(0)
    idx_wait(1)
    out_wait(0)
    out_wait(1)


_sc_scores = functools.partial(
    pl.kernel,
    mesh=plsc.VectorSubcoreMesh(core_axis_name="c", subcore_axis_name="s"),
    compiler_params=pltpu.CompilerParams(
        needs_layout_passes=False, use_tc_tiling_on_sc=False),
    out_type=jax.ShapeDtypeStruct((_N_EDGES,), jnp.float32),
    scratch_types=[
        pltpu.VMEM((_B,), jnp.int32),
        pltpu.VMEM((_B,), jnp.int32),
        pltpu.VMEM((_B,), jnp.int32),
        pltpu.VMEM((_B,), jnp.int32),
        pltpu.VMEM((_B, _DW), jnp.int32),
        pltpu.VMEM((_B, _DW), jnp.int32),
        pltpu.VMEM((_B, _DW), jnp.int32),
        pltpu.VMEM((_B, _DW), jnp.int32),
        pltpu.VMEM((_B + _L,), jnp.float32),
        pltpu.VMEM((_B + _L,), jnp.float32),
        pltpu.SemaphoreType.DMA,
        pltpu.SemaphoreType.DMA,
        pltpu.SemaphoreType.DMA,
        pltpu.SemaphoreType.DMA,
        pltpu.SemaphoreType.DMA,
        pltpu.SemaphoreType.DMA,
    ],
)(_sc_scores_body)


_FLAT_COLS = 128
_FLAT_ROWS = _N_EDGES // _FLAT_COLS
_POS_ROWS = _N_POS // _FLAT_COLS


def _tc_reduce_body(y_ref, o_ref):
    row = lax.broadcasted_iota(jnp.int32, (_FLAT_ROWS, _FLAT_COLS), 0)
    sign = jnp.where(row < _POS_ROWS, -1.0, 1.0).astype(jnp.float32)
    y = y_ref[...] * sign
    sp = jnp.maximum(y, 0.0) + jnp.log(1.0 + jnp.exp(-jnp.abs(y)))
    o_ref[0, 0] = jnp.sum(sp) * (1.0 / _N_POS)


def kernel(embeddings, pos_edges, neg_edges):
    edges = jnp.concatenate(
        [pos_edges.astype(jnp.int32), neg_edges.astype(jnp.int32)], axis=1)
    edges_flat = edges.reshape(-1)
    emb_words = jax.lax.bitcast_convert_type(
        embeddings.astype(jnp.bfloat16).reshape(_N_NODES, _DW, 2),
        jnp.int32)
    scores = _sc_scores(emb_words, edges_flat)
    y = scores.reshape(_FLAT_ROWS, _FLAT_COLS)
    loss = pl.pallas_call(
        _tc_reduce_body,
        out_shape=jax.ShapeDtypeStruct((1, 1), jnp.float32),
        out_specs=pl.BlockSpec(memory_space=pltpu.SMEM),
    )(y)
    return loss[0, 0]
